# baseline (device time: 80034 ns/iter reference)
import jax
import jax.numpy as jnp
from jax import lax
from jax.experimental import pallas as pl
from jax.experimental.pallas import tpu as pltpu

N_DEV = 4
SQ = 128
SKV_LOCAL = 32768
D = 1024
HQ = 8
HKV = 2
DH = 128
GROUP = HQ // HKV
SCALE = 0.08838834764831843
CHUNK = 2048
N_CHUNKS = SKV_LOCAL // CHUNK
PROWS = SQ + 1


def _attn_body(x_ref, wq_ref, k_hbm, v_hbm, out_ref,
               q_ref, kv_ref, acc_ref, l_ref, dma_sems):
    copies = {}

    def start_chunk(slot, j):
        for st in range(4):
            src = k_hbm if st < 2 else v_hbm
            g = st % 2
            c = pltpu.make_async_copy(
                src.at[0, pl.ds(j * CHUNK, CHUNK), g, :],
                kv_ref.at[slot, st],
                dma_sems.at[slot, st],
            )
            c.start()
            copies[(slot, st)] = c

    start_chunk(0, 0)

    xb = x_ref[...].astype(jnp.bfloat16)
    wqb = wq_ref[...].astype(jnp.bfloat16)
    for h in range(HQ):
        qh = jnp.dot(xb, wqb[:, h * DH:(h + 1) * DH],
                     preferred_element_type=jnp.float32)
        q_ref[h * SQ:(h + 1) * SQ, :] = (qh * SCALE).astype(jnp.bfloat16)

    acc_ref[...] = jnp.zeros((HQ * SQ, DH), jnp.float32)
    l_ref[...] = jnp.zeros((HQ * SQ, 1), jnp.float32)

    ones_bf = jnp.ones((CHUNK, DH), jnp.bfloat16)
    for j in range(N_CHUNKS):
        slot = j % 2
        if j + 1 < N_CHUNKS:
            start_chunk(1 - slot, j + 1)
        for st in range(4):
            copies[(slot, st)].wait()
        for g in range(HKV):
            rows = slice(g * GROUP * SQ, (g + 1) * GROUP * SQ)
            k_g = kv_ref[slot, g].astype(jnp.bfloat16)
            v_g = kv_ref[slot, 2 + g].astype(jnp.bfloat16)
            s = lax.dot_general(
                q_ref[rows, :], k_g, (((1,), (1,)), ((), ())),
                preferred_element_type=jnp.float32,
            )
            p_bf = jnp.exp(s).astype(jnp.bfloat16)
            pv = lax.dot_general(
                p_bf, v_g, (((1,), (0,)), ((), ())),
                preferred_element_type=jnp.float32,
            )
            acc_ref[rows, :] = acc_ref[rows, :] + pv
            lcol = lax.dot_general(
                p_bf, ones_bf, (((1,), (0,)), ((), ())),
                preferred_element_type=jnp.float32,
            )
            l_ref[rows, :] = l_ref[rows, :] + lcol[:, :1]

    out_ref[:, :SQ, :] = acc_ref[...].astype(jnp.bfloat16).reshape(HQ, SQ, DH)
    out_ref[:, SQ, :] = l_ref[...][:, 0].reshape(HQ, SQ).astype(jnp.bfloat16)


def _combine_body(part_ref, wo_ref, out_ref, comm_ref, attn_ref,
                  send_sems, recv_sems):
    my = lax.axis_index("i")
    p1 = jnp.bitwise_xor(my, 1)
    p2 = jnp.bitwise_xor(my, 2)

    barrier = pltpu.get_barrier_semaphore()
    pl.semaphore_signal(barrier, inc=1, device_id=(p1,),
                        device_id_type=pl.DeviceIdType.MESH)
    pl.semaphore_signal(barrier, inc=1, device_id=(p2,),
                        device_id_type=pl.DeviceIdType.MESH)
    pl.semaphore_wait(barrier, 2)

    comm_ref[0] = part_ref[...]

    r1 = pltpu.make_async_remote_copy(
        src_ref=comm_ref.at[0], dst_ref=comm_ref.at[1],
        send_sem=send_sems.at[0], recv_sem=recv_sems.at[0],
        device_id=(p1,), device_id_type=pl.DeviceIdType.MESH,
    )
    r1.start()
    r1.wait()

    acc = (comm_ref[0, :, :SQ, :].astype(jnp.float32)
           + comm_ref[1, :, :SQ, :].astype(jnp.float32))
    l = (comm_ref[0, :, SQ, :].astype(jnp.float32)
         + comm_ref[1, :, SQ, :].astype(jnp.float32))
    comm_ref[2, :, :SQ, :] = acc.astype(jnp.bfloat16)
    comm_ref[2, :, SQ, :] = l.astype(jnp.bfloat16)

    r2 = pltpu.make_async_remote_copy(
        src_ref=comm_ref.at[2], dst_ref=comm_ref.at[3],
        send_sem=send_sems.at[1], recv_sem=recv_sems.at[1],
        device_id=(p2,), device_id_type=pl.DeviceIdType.MESH,
    )
    r2.start()
    r2.wait()

    acc = acc + comm_ref[3, :, :SQ, :].astype(jnp.float32)
    l = l + comm_ref[3, :, SQ, :].astype(jnp.float32)

    o = acc / l[:, :, None]
    for h in range(HQ):
        attn_ref[:, h * DH:(h + 1) * DH] = o[h].astype(jnp.bfloat16)
    out_ref[...] = jnp.dot(
        attn_ref[...], wo_ref[...].astype(jnp.bfloat16),
        preferred_element_type=jnp.float32,
    )


def kernel(x, Wq, Wo, K_ext, V_ext):
    x2 = x.reshape(SQ, D)

    partial = pl.pallas_call(
        _attn_body,
        in_specs=[
            pl.BlockSpec(memory_space=pltpu.VMEM),
            pl.BlockSpec(memory_space=pltpu.VMEM),
            pl.BlockSpec(memory_space=pl.ANY),
            pl.BlockSpec(memory_space=pl.ANY),
        ],
        out_specs=pl.BlockSpec(memory_space=pltpu.VMEM),
        out_shape=jax.ShapeDtypeStruct((HQ, PROWS, DH), jnp.bfloat16),
        scratch_shapes=[
            pltpu.VMEM((HQ * SQ, DH), jnp.bfloat16),
            pltpu.VMEM((2, 4, CHUNK, DH), jnp.float32),
            pltpu.VMEM((HQ * SQ, DH), jnp.float32),
            pltpu.VMEM((HQ * SQ, 1), jnp.float32),
            pltpu.SemaphoreType.DMA((2, 4)),
        ],
    )(x2, Wq, K_ext, V_ext)

    out = pl.pallas_call(
        _combine_body,
        in_specs=[
            pl.BlockSpec(memory_space=pltpu.VMEM),
            pl.BlockSpec(memory_space=pltpu.VMEM),
        ],
        out_specs=pl.BlockSpec(memory_space=pltpu.VMEM),
        out_shape=jax.ShapeDtypeStruct((SQ, D), jnp.float32),
        scratch_shapes=[
            pltpu.VMEM((4, HQ, PROWS, DH), jnp.bfloat16),
            pltpu.VMEM((SQ, D), jnp.bfloat16),
            pltpu.SemaphoreType.DMA((2,)),
            pltpu.SemaphoreType.DMA((2,)),
        ],
        compiler_params=pltpu.CompilerParams(collective_id=0),
    )(partial, Wo)

    return out.reshape(1, SQ, D)


# device time: 68801 ns/iter; 1.1633x vs baseline; 1.1633x over previous
import jax
import jax.numpy as jnp
from jax import lax
from jax.experimental import pallas as pl
from jax.experimental.pallas import tpu as pltpu

N_DEV = 4
SQ = 128
SKV_LOCAL = 32768
D = 1024
HQ = 8
HKV = 2
DH = 128
GROUP = HQ // HKV
ROWS_G = GROUP * SQ
SCALE = 0.08838834764831843
CHUNK = 2048
N_CHUNKS = SKV_LOCAL // CHUNK
N_VCHUNKS = HKV * N_CHUNKS
PROWS = SQ + 1
COMBINE_VJ = N_CHUNKS + 3


def _body(x_ref, wq_ref, wo_ref, k_hbm, v_hbm, out_ref,
          q_ref, kv_ref, acc_ref, l_ref, attn_ref, comm_ref,
          dma_sems, send_sems, recv_sems):
    my = lax.axis_index("i")
    p1 = jnp.bitwise_xor(my, 1)
    p2 = jnp.bitwise_xor(my, 2)

    barrier = pltpu.get_barrier_semaphore()
    pl.semaphore_signal(barrier, inc=1, device_id=(p1,),
                        device_id_type=pl.DeviceIdType.MESH)
    pl.semaphore_signal(barrier, inc=1, device_id=(p2,),
                        device_id_type=pl.DeviceIdType.MESH)
    pl.semaphore_wait(barrier, 2)

    copies = {}

    def start_vchunk(slot, vj):
        g, jj = divmod(vj, N_CHUNKS)
        for st, src in ((0, k_hbm), (1, v_hbm)):
            c = pltpu.make_async_copy(
                src.at[0, pl.ds(jj * CHUNK, CHUNK), g, :],
                kv_ref.at[slot, st],
                dma_sems.at[slot, st],
            )
            c.start()
            copies[(slot, st)] = c

    start_vchunk(0, 0)

    xb = x_ref[...].astype(jnp.bfloat16)
    wqb = wq_ref[...].astype(jnp.bfloat16)
    for h in range(HQ):
        qh = jnp.dot(xb, wqb[:, h * DH:(h + 1) * DH],
                     preferred_element_type=jnp.float32)
        q_ref[h * SQ:(h + 1) * SQ, :] = (qh * SCALE).astype(jnp.bfloat16)

    acc_ref[...] = jnp.zeros((HQ * SQ, DH), jnp.float32)
    l_ref[...] = jnp.zeros((HQ * SQ, 1), jnp.float32)

    rdmas = {}

    def pack_and_start_r1(g):
        rows = slice(g * ROWS_G, (g + 1) * ROWS_G)
        comm_ref[g, 0, :, :SQ, :] = (
            acc_ref[rows, :].astype(jnp.bfloat16).reshape(GROUP, SQ, DH))
        comm_ref[g, 0, :, SQ, :] = (
            l_ref[rows, :][:, 0].reshape(GROUP, SQ).astype(jnp.bfloat16))
        r = pltpu.make_async_remote_copy(
            src_ref=comm_ref.at[g, 0], dst_ref=comm_ref.at[g, 1],
            send_sem=send_sems.at[g, 0], recv_sem=recv_sems.at[g, 0],
            device_id=(p1,), device_id_type=pl.DeviceIdType.MESH,
        )
        r.start()
        rdmas[(g, 1)] = r

    def combine1_and_start_r2(g):
        rdmas[(g, 1)].wait()
        a = (comm_ref[g, 0, :, :SQ, :].astype(jnp.float32)
             + comm_ref[g, 1, :, :SQ, :].astype(jnp.float32))
        l = (comm_ref[g, 0, :, SQ, :].astype(jnp.float32)
             + comm_ref[g, 1, :, SQ, :].astype(jnp.float32))
        comm_ref[g, 2, :, :SQ, :] = a.astype(jnp.bfloat16)
        comm_ref[g, 2, :, SQ, :] = l.astype(jnp.bfloat16)
        r = pltpu.make_async_remote_copy(
            src_ref=comm_ref.at[g, 2], dst_ref=comm_ref.at[g, 3],
            send_sem=send_sems.at[g, 1], recv_sem=recv_sems.at[g, 1],
            device_id=(p2,), device_id_type=pl.DeviceIdType.MESH,
        )
        r.start()
        rdmas[(g, 2)] = r
        return a, l

    def finish(g, a, l):
        rdmas[(g, 2)].wait()
        a = a + comm_ref[g, 3, :, :SQ, :].astype(jnp.float32)
        l = l + comm_ref[g, 3, :, SQ, :].astype(jnp.float32)
        o = a / l[:, :, None]
        for hh in range(GROUP):
            h = g * GROUP + hh
            attn_ref[:, h * DH:(h + 1) * DH] = o[hh].astype(jnp.bfloat16)

    carry0 = None
    for vj in range(N_VCHUNKS):
        g = vj // N_CHUNKS
        slot = vj % 2
        if vj + 1 < N_VCHUNKS:
            start_vchunk(1 - slot, vj + 1)
        copies[(slot, 0)].wait()
        copies[(slot, 1)].wait()

        rows = slice(g * ROWS_G, (g + 1) * ROWS_G)
        k_g = kv_ref[slot, 0].astype(jnp.bfloat16)
        v_g = kv_ref[slot, 1].astype(jnp.bfloat16)
        s = lax.dot_general(
            q_ref[rows, :], k_g, (((1,), (1,)), ((), ())),
            preferred_element_type=jnp.float32,
        )
        p_bf = jnp.exp(s).astype(jnp.bfloat16)
        l_ref[rows, :] = l_ref[rows, :] + jnp.sum(
            p_bf, axis=1, keepdims=True, dtype=jnp.float32)
        pv = lax.dot_general(
            p_bf, v_g, (((1,), (0,)), ((), ())),
            preferred_element_type=jnp.float32,
        )
        acc_ref[rows, :] = acc_ref[rows, :] + pv

        if vj == N_CHUNKS - 1:
            pack_and_start_r1(0)
        if vj == COMBINE_VJ:
            carry0 = combine1_and_start_r2(0)

    pack_and_start_r1(1)
    finish(0, *carry0)
    carry1 = combine1_and_start_r2(1)
    finish(1, *carry1)

    out_ref[...] = jnp.dot(
        attn_ref[...], wo_ref[...].astype(jnp.bfloat16),
        preferred_element_type=jnp.float32,
    )


def kernel(x, Wq, Wo, K_ext, V_ext):
    x2 = x.reshape(SQ, D)

    out = pl.pallas_call(
        _body,
        in_specs=[
            pl.BlockSpec(memory_space=pltpu.VMEM),
            pl.BlockSpec(memory_space=pltpu.VMEM),
            pl.BlockSpec(memory_space=pltpu.VMEM),
            pl.BlockSpec(memory_space=pl.ANY),
            pl.BlockSpec(memory_space=pl.ANY),
        ],
        out_specs=pl.BlockSpec(memory_space=pltpu.VMEM),
        out_shape=jax.ShapeDtypeStruct((SQ, D), jnp.float32),
        scratch_shapes=[
            pltpu.VMEM((HQ * SQ, DH), jnp.bfloat16),
            pltpu.VMEM((2, 2, CHUNK, DH), jnp.float32),
            pltpu.VMEM((HQ * SQ, DH), jnp.float32),
            pltpu.VMEM((HQ * SQ, 1), jnp.float32),
            pltpu.VMEM((SQ, D), jnp.bfloat16),
            pltpu.VMEM((HKV, 4, GROUP, PROWS, DH), jnp.bfloat16),
            pltpu.SemaphoreType.DMA((2, 2)),
            pltpu.SemaphoreType.DMA((HKV, 2)),
            pltpu.SemaphoreType.DMA((HKV, 2)),
        ],
        compiler_params=pltpu.CompilerParams(collective_id=0),
    )(x2, Wq, Wo, K_ext, V_ext)

    return out.reshape(1, SQ, D)


# device time: 63471 ns/iter; 1.2610x vs baseline; 1.0840x over previous
import jax
import jax.numpy as jnp
from jax import lax
from jax.experimental import pallas as pl
from jax.experimental.pallas import tpu as pltpu

N_DEV = 4
SQ = 128
SKV_LOCAL = 32768
D = 1024
HQ = 8
HKV = 2
DH = 128
GROUP = HQ // HKV
ROWS_G = GROUP * SQ
SCALE = 0.08838834764831843
CHUNK = 2048
N_CHUNKS = SKV_LOCAL // CHUNK
PROWS = SQ + 1


def _attn_body(x_ref, wq_ref, k_hbm, v_hbm, out_ref,
               q_ref, kv_ref, acc_ref, l_ref, dma_sems):
    copies = {}

    def start_chunk(slot, j):
        for st in range(4):
            src = k_hbm if st < 2 else v_hbm
            g = st % 2
            c = pltpu.make_async_copy(
                src.at[0, pl.ds(j * CHUNK, CHUNK), g, :],
                kv_ref.at[slot, st],
                dma_sems.at[slot, st],
            )
            c.start()
            copies[(slot, st)] = c

    start_chunk(0, 0)

    xb = x_ref[...].astype(jnp.bfloat16)
    wqb = wq_ref[...].astype(jnp.bfloat16)
    for h in range(HQ):
        qh = jnp.dot(xb, wqb[:, h * DH:(h + 1) * DH],
                     preferred_element_type=jnp.float32)
        q_ref[h * SQ:(h + 1) * SQ, :] = (qh * SCALE).astype(jnp.bfloat16)

    acc_ref[...] = jnp.zeros((HQ * SQ, DH), jnp.float32)
    l_ref[...] = jnp.zeros((HQ * SQ, 1), jnp.float32)

    for j in range(N_CHUNKS):
        slot = j % 2
        if j + 1 < N_CHUNKS:
            start_chunk(1 - slot, j + 1)
        for st in range(4):
            copies[(slot, st)].wait()
        for g in range(HKV):
            rows = slice(g * ROWS_G, (g + 1) * ROWS_G)
            k_g = kv_ref[slot, g].astype(jnp.bfloat16)
            v_g = kv_ref[slot, 2 + g].astype(jnp.bfloat16)
            s = lax.dot_general(
                q_ref[rows, :], k_g, (((1,), (1,)), ((), ())),
                preferred_element_type=jnp.float32,
            )
            p = jnp.exp(s)
            l_ref[rows, :] = l_ref[rows, :] + jnp.sum(p, axis=1, keepdims=True)
            pv = lax.dot_general(
                p.astype(jnp.bfloat16), v_g, (((1,), (0,)), ((), ())),
                preferred_element_type=jnp.float32,
            )
            acc_ref[rows, :] = acc_ref[rows, :] + pv

    out_ref[:, :SQ, :] = acc_ref[...].astype(jnp.bfloat16).reshape(HQ, SQ, DH)
    out_ref[:, SQ, :] = l_ref[...][:, 0].reshape(HQ, SQ).astype(jnp.bfloat16)


def _combine_body(part_ref, wo_ref, out_ref, comm_ref, attn_ref,
                  send_sems, recv_sems):
    my = lax.axis_index("i")
    p1 = jnp.bitwise_xor(my, 1)
    p2 = jnp.bitwise_xor(my, 2)

    barrier = pltpu.get_barrier_semaphore()
    pl.semaphore_signal(barrier, inc=1, device_id=(p1,),
                        device_id_type=pl.DeviceIdType.MESH)
    pl.semaphore_signal(barrier, inc=1, device_id=(p2,),
                        device_id_type=pl.DeviceIdType.MESH)
    pl.semaphore_wait(barrier, 2)

    comm_ref[0] = part_ref[...]

    r1 = pltpu.make_async_remote_copy(
        src_ref=comm_ref.at[0], dst_ref=comm_ref.at[1],
        send_sem=send_sems.at[0], recv_sem=recv_sems.at[0],
        device_id=(p1,), device_id_type=pl.DeviceIdType.MESH,
    )
    r1.start()
    r1.wait()

    acc = (comm_ref[0, :, :SQ, :].astype(jnp.float32)
           + comm_ref[1, :, :SQ, :].astype(jnp.float32))
    l = (comm_ref[0, :, SQ, :].astype(jnp.float32)
         + comm_ref[1, :, SQ, :].astype(jnp.float32))
    comm_ref[2, :, :SQ, :] = acc.astype(jnp.bfloat16)
    comm_ref[2, :, SQ, :] = l.astype(jnp.bfloat16)

    r2 = pltpu.make_async_remote_copy(
        src_ref=comm_ref.at[2], dst_ref=comm_ref.at[3],
        send_sem=send_sems.at[1], recv_sem=recv_sems.at[1],
        device_id=(p2,), device_id_type=pl.DeviceIdType.MESH,
    )
    r2.start()
    r2.wait()

    acc = acc + comm_ref[3, :, :SQ, :].astype(jnp.float32)
    l = l + comm_ref[3, :, SQ, :].astype(jnp.float32)

    o = acc / l[:, :, None]
    for h in range(HQ):
        attn_ref[:, h * DH:(h + 1) * DH] = o[h].astype(jnp.bfloat16)
    out_ref[...] = jnp.dot(
        attn_ref[...], wo_ref[...].astype(jnp.bfloat16),
        preferred_element_type=jnp.float32,
    )


def kernel(x, Wq, Wo, K_ext, V_ext):
    x2 = x.reshape(SQ, D)

    partial = pl.pallas_call(
        _attn_body,
        in_specs=[
            pl.BlockSpec(memory_space=pltpu.VMEM),
            pl.BlockSpec(memory_space=pltpu.VMEM),
            pl.BlockSpec(memory_space=pl.ANY),
            pl.BlockSpec(memory_space=pl.ANY),
        ],
        out_specs=pl.BlockSpec(memory_space=pltpu.VMEM),
        out_shape=jax.ShapeDtypeStruct((HQ, PROWS, DH), jnp.bfloat16),
        scratch_shapes=[
            pltpu.VMEM((HQ * SQ, DH), jnp.bfloat16),
            pltpu.VMEM((2, 4, CHUNK, DH), jnp.float32),
            pltpu.VMEM((HQ * SQ, DH), jnp.float32),
            pltpu.VMEM((HQ * SQ, 1), jnp.float32),
            pltpu.SemaphoreType.DMA((2, 4)),
        ],
    )(x2, Wq, K_ext, V_ext)

    out = pl.pallas_call(
        _combine_body,
        in_specs=[
            pl.BlockSpec(memory_space=pltpu.VMEM),
            pl.BlockSpec(memory_space=pltpu.VMEM),
        ],
        out_specs=pl.BlockSpec(memory_space=pltpu.VMEM),
        out_shape=jax.ShapeDtypeStruct((SQ, D), jnp.float32),
        scratch_shapes=[
            pltpu.VMEM((4, HQ, PROWS, DH), jnp.bfloat16),
            pltpu.VMEM((SQ, D), jnp.bfloat16),
            pltpu.SemaphoreType.DMA((2,)),
            pltpu.SemaphoreType.DMA((2,)),
        ],
        compiler_params=pltpu.CompilerParams(collective_id=0),
    )(partial, Wo)

    return out.reshape(1, SQ, D)
